# Initial kernel scaffold; baseline (speedup 1.0000x reference)
import jax
import jax.numpy as jnp
from jax.experimental import pallas as pl


def kernel(x, edge_index, edge_type, batch, w1, root1, b1, w2, root2, b2, g1w, g1b, g1a, g2w, g2b, g2a, lin_w, lin_b):
    raise NotImplementedError("write your pallas kernel here")


# speedup vs baseline: 5.5916x; 5.5916x over previous
import functools

import jax
import jax.numpy as jnp
from jax import lax
from jax.experimental import pallas as pl
from jax.experimental.pallas import tpu as pltpu
from jax.experimental.pallas import tpu_sc as plsc

F32 = jnp.float32
HI = jax.lax.Precision.HIGHEST
I32 = jnp.int32

NC = 2
NS = 16
LANES = 16
NW = NC * NS

G = 64
PAD_G = 64.0

NB = 1024
NPAD = 50176
NBLK = NPAD // NB

E_PAD = 819200
EPW = E_PAD // NW
CK = 1600
KG = 128
CB = CK + KG + LANES

RID_CAP = 4 * NPAD
NP = 9
SCROWS = 12160
ACC_ROWS = 12288
TROWS = ACC_ROWS // NS
ZROWS = 64
CROWS = SCROWS // NS
RTOT_E = NP * NC * SCROWS


def _edge_scalar_sc(xf, src_p, rid_p):
    mesh = plsc.VectorSubcoreMesh(
        core_axis_name="c", subcore_axis_name="s", num_cores=NC,
        num_subcores=NS)
    zslice = RID_CAP // NS

    @functools.partial(
        pl.kernel,
        out_type=(jax.ShapeDtypeStruct((NC, RID_CAP), F32),
                  jax.ShapeDtypeStruct((NC, RID_CAP), F32)),
        mesh=mesh,
        scratch_types=[
            pltpu.VMEM((CK,), I32),
            pltpu.VMEM((CK,), I32),
            pltpu.VMEM((CK,), F32),
            pltpu.VMEM((CK,), F32),
            pltpu.VMEM_SHARED((RID_CAP,), F32),
            pltpu.VMEM_SHARED((RID_CAP,), F32),
            pltpu.SemaphoreType.DMA,
        ],
    )
    def k(x_hbm, src_hbm, rid_hbm, z_hbm, outv_hbm, outc_hbm,
          src_v, rid_v, val_v, ones_v, accv, accc, sem):
        c = lax.axis_index("c")
        s = lax.axis_index("s")
        wid = s * NC + c

        pltpu.sync_copy(z_hbm.at[pl.ds(s * zslice, zslice)],
                        accv.at[pl.ds(s * zslice, zslice)])
        pltpu.sync_copy(z_hbm.at[pl.ds(s * zslice, zslice)],
                        accc.at[pl.ds(s * zslice, zslice)])

        def fill(i, _):
            ones_v[pl.ds(i * LANES, LANES)] = jnp.full((LANES,), 1.0, F32)
            return 0
        lax.fori_loop(0, CK // LANES, fill, 0)
        plsc.subcore_barrier()

        def step(i, _):
            base = wid * EPW + i * CK
            pltpu.sync_copy(src_hbm.at[pl.ds(base, CK)], src_v)
            pltpu.sync_copy(rid_hbm.at[pl.ds(base, CK)], rid_v)
            pltpu.async_copy(x_hbm.at[src_v], val_v, sem).wait()
            pltpu.sync_copy(val_v, accv.at[rid_v], add=True)
            pltpu.sync_copy(ones_v, accc.at[rid_v], add=True)
            return 0
        lax.fori_loop(0, EPW // CK, step, 0)
        plsc.subcore_barrier()

        pltpu.sync_copy(accv.at[pl.ds(s * zslice, zslice)],
                        outv_hbm.at[c, pl.ds(s * zslice, zslice)])
        pltpu.sync_copy(accc.at[pl.ds(s * zslice, zslice)],
                        outc_hbm.at[c, pl.ds(s * zslice, zslice)])

    zeros = jnp.zeros((RID_CAP,), F32)
    return k(xf, src_p, rid_p, zeros)


def _edge_rows_sc(h1, src_p, rid_p):
    mesh = plsc.VectorSubcoreMesh(
        core_axis_name="c", subcore_axis_name="s", num_cores=NC,
        num_subcores=NS)

    @functools.partial(
        pl.kernel,
        out_type=jax.ShapeDtypeStruct((RTOT_E, 128), F32),
        mesh=mesh,
        scratch_types=[
            pltpu.VMEM((CB,), I32),
            pltpu.VMEM((CB,), I32),
            pltpu.VMEM((CK,), I32),
            pltpu.VMEM((CK,), I32),
            pltpu.VMEM((KG,), I32),
            pltpu.VMEM((KG,), I32),
            pltpu.VMEM((KG, 128), F32),
            pltpu.VMEM((ZROWS, 128), F32),
            pltpu.VMEM_SHARED((ACC_ROWS, 128), F32),
            pltpu.SemaphoreType.DMA,
        ],
        compiler_params=pltpu.CompilerParams(needs_layout_passes=False),
    )
    def k(h_hbm, src_hbm, rid_hbm, out_hbm,
          src_c, lid_c, rid_v, src_v, idxg, idxs, rows_v, zrows, acc, sem):
        c = lax.axis_index("c")
        s = lax.axis_index("s")
        ept = E_PAD // NS

        def zfill(i, _):
            for kk in range(128 // LANES):
                zrows[i, pl.ds(kk * LANES, LANES)] = (
                    jnp.zeros((LANES,), F32))
            return 0
        lax.fori_loop(0, ZROWS, zfill, 0)
        iot = lax.iota(I32, LANES)

        def drain(t, _):
            def cp(j, _):
                idxg[pl.ds(j * LANES, LANES)] = (
                    src_c[pl.ds(t * KG + j * LANES, LANES)])
                idxs[pl.ds(j * LANES, LANES)] = (
                    lid_c[pl.ds(t * KG + j * LANES, LANES)])
                return 0
            lax.fori_loop(0, KG // LANES, cp, 0)
            pltpu.async_copy(h_hbm.at[idxg], rows_v, sem).wait()
            pltpu.sync_copy(rows_v, acc.at[idxs], add=True)
            return 0

        def pass_body(p, _):
            r0 = (p * NC + c) * SCROWS

            def zero_acc(kk, _):
                pltpu.sync_copy(
                    zrows, acc.at[pl.ds(s * TROWS + kk * ZROWS, ZROWS)])
                return 0
            lax.fori_loop(0, TROWS // ZROWS, zero_acc, 0)
            plsc.subcore_barrier()

            def chunk(i, off):
                base = s * ept + i * CK
                pltpu.sync_copy(src_hbm.at[pl.ds(base, CK)], src_v)
                pltpu.sync_copy(rid_hbm.at[pl.ds(base, CK)], rid_v)

                def vec(j, off):
                    rv = rid_v[pl.ds(j * LANES, LANES)]
                    sv = src_v[pl.ds(j * LANES, LANES)]
                    lid = rv - r0
                    m = (lid >= 0) & (lid < SCROWS)
                    cs = plsc.cumsum(jnp.where(m, 1.0, 0.0))
                    pos = off + cs.astype(I32) - 1
                    plsc.store_scatter(lid_c, [pos], lid, mask=m)
                    plsc.store_scatter(src_c, [pos], sv, mask=m)
                    return off + cs[LANES - 1].astype(I32)
                off = lax.fori_loop(0, CK // LANES, vec, off)

                nfull = off // KG
                lax.fori_loop(0, nfull, drain, 0)
                rem = off - nfull * KG

                def mv(kk, _):
                    lid_c[pl.ds(kk * LANES, LANES)] = (
                        lid_c[pl.ds(nfull * KG + kk * LANES, LANES)])
                    src_c[pl.ds(kk * LANES, LANES)] = (
                        src_c[pl.ds(nfull * KG + kk * LANES, LANES)])
                    return 0
                lax.fori_loop(0, (rem + LANES - 1) // LANES, mv, 0)
                return rem
            off = lax.fori_loop(0, (E_PAD // NS) // CK, chunk,
                                jnp.zeros((), I32))

            for kk in range(KG // LANES):
                pos = off + kk * LANES + iot
                plsc.store_scatter(
                    lid_c, [pos], jnp.full((LANES,), SCROWS, I32))
                plsc.store_scatter(
                    src_c, [pos], jnp.zeros((LANES,), I32))
            lax.fori_loop(0, (off + KG - 1) // KG, drain, 0)
            plsc.subcore_barrier()

            pltpu.sync_copy(acc.at[pl.ds(s * CROWS, CROWS)],
                            out_hbm.at[pl.ds(r0 + s * CROWS, CROWS)])
            plsc.subcore_barrier()
            return 0
        lax.fori_loop(0, NP, pass_body, 0)

    return k(h1, src_p, rid_p)


def _iota_row(width):
    return lax.broadcasted_iota(I32, (1, width), 1).astype(F32)


def _tc_layer1(xp, svr, scr, bcol, brow, w1m, r1, b1r):
    def body(x_ref, sv_ref, sc_ref, bc_ref, br_ref, w_ref, r_ref, b_ref,
             z_ref, s1_ref, s2_ref, cg_ref):
        i = pl.program_id(0)
        xb = x_ref[...]
        sv = sv_ref[0] + sv_ref[1]
        scnt = sc_ref[0] + sc_ref[1]
        mean = sv / jnp.maximum(scnt, 1.0)
        z = (xb * r_ref[...]
             + jnp.dot(mean, w_ref[...], preferred_element_type=F32, precision=HI)
             + b_ref[...])
        z_ref[...] = z
        oht = (br_ref[...] == lax.broadcasted_iota(I32, (G, NB), 0).astype(F32)
               ).astype(F32)
        s1c = jnp.dot(oht, z, preferred_element_type=F32, precision=HI)
        s2c = jnp.dot(oht, z * z, preferred_element_type=F32, precision=HI)
        cgc = jnp.dot(oht, jnp.ones_like(z), preferred_element_type=F32, precision=HI)

        @pl.when(i == 0)
        def _():
            s1_ref[...] = jnp.zeros_like(s1_ref)
            s2_ref[...] = jnp.zeros_like(s2_ref)
            cg_ref[...] = jnp.zeros_like(cg_ref)
        s1_ref[...] += s1c
        s2_ref[...] += s2c
        cg_ref[...] += cgc

    full = lambda shape: pl.BlockSpec(shape, lambda i: tuple(0 for _ in shape))
    return pl.pallas_call(
        body,
        grid=(NBLK,),
        in_specs=[
            pl.BlockSpec((NB, 1), lambda i: (i, 0)),
            pl.BlockSpec((2, NB, 4), lambda i: (0, i, 0)),
            pl.BlockSpec((2, NB, 4), lambda i: (0, i, 0)),
            pl.BlockSpec((NB, 1), lambda i: (i, 0)),
            pl.BlockSpec((1, NB), lambda i: (0, i)),
            full((4, 128)),
            full((1, 128)),
            full((1, 128)),
        ],
        out_specs=[
            pl.BlockSpec((NB, 128), lambda i: (i, 0)),
            full((G, 128)),
            full((G, 128)),
            full((G, 128)),
        ],
        out_shape=[
            jax.ShapeDtypeStruct((NPAD, 128), F32),
            jax.ShapeDtypeStruct((G, 128), F32),
            jax.ShapeDtypeStruct((G, 128), F32),
            jax.ShapeDtypeStruct((G, 128), F32),
        ],
    )(xp, svr, scr, bcol, brow, w1m, r1, b1r)


def _tc_norm_params(s1, s2, cg, gw, gb, ga, width):
    def body(s1_ref, s2_ref, cg_ref, gw_ref, gb_ref, ga_ref, a_ref, b_ref):
        c = jnp.maximum(cg_ref[...], 1.0)
        m = s1_ref[...] / c
        a = ga_ref[...]
        var = s2_ref[...] / c - (2.0 * a - a * a) * m * m
        aa = gw_ref[...] * lax.rsqrt(var + 1e-5)
        a_ref[...] = aa
        b_ref[...] = gb_ref[...] - aa * a * m

    full = lambda shape: pl.BlockSpec(shape, lambda: tuple(0 for _ in shape))
    return pl.pallas_call(
        body,
        in_specs=[full((G, width))] * 3 + [full((1, width))] * 3,
        out_specs=[full((G, width))] * 2,
        out_shape=[jax.ShapeDtypeStruct((G, width), F32)] * 2,
    )(s1, s2, cg, gw, gb, ga)


def _tc_apply_relu(z1, bcol, a1, b1n):
    def body(z_ref, bc_ref, a_ref, b_ref, h_ref):
        oh = (bc_ref[...] == _iota_row(G)).astype(F32)
        an = jnp.dot(oh, a_ref[...], preferred_element_type=F32, precision=HI)
        bn = jnp.dot(oh, b_ref[...], preferred_element_type=F32, precision=HI)
        h_ref[...] = jnp.maximum(an * z_ref[...] + bn, 0.0)

    full = lambda shape: pl.BlockSpec(shape, lambda i: tuple(0 for _ in shape))
    return pl.pallas_call(
        body,
        grid=(NBLK,),
        in_specs=[
            pl.BlockSpec((NB, 128), lambda i: (i, 0)),
            pl.BlockSpec((NB, 1), lambda i: (i, 0)),
            full((G, 128)),
            full((G, 128)),
        ],
        out_specs=pl.BlockSpec((NB, 128), lambda i: (i, 0)),
        out_shape=jax.ShapeDtypeStruct((NPAD, 128), F32),
    )(z1, bcol, a1, b1n)


def _tc_layer2(h1, accr, scr, bcol, brow, expand, r2, w2s, b2r):
    def body(h_ref, acc_ref, sc_ref, bc_ref, br_ref, e_ref, r_ref, w_ref,
             b_ref, z_ref, s1_ref, s2_ref, cg_ref):
        i = pl.program_id(0)
        cnt = sc_ref[0] + sc_ref[1]
        recip = 1.0 / jnp.maximum(cnt, 1.0)
        rb = jnp.dot(recip, e_ref[...], preferred_element_type=F32, precision=HI)
        mean = acc_ref[...] * rb
        z = (jnp.dot(h_ref[...], r_ref[...], preferred_element_type=F32, precision=HI)
             + jnp.dot(mean, w_ref[...], preferred_element_type=F32, precision=HI)
             + b_ref[...])
        z_ref[...] = z
        oht = (br_ref[...] == lax.broadcasted_iota(I32, (G, NB), 0).astype(F32)
               ).astype(F32)
        s1c = jnp.dot(oht, z, preferred_element_type=F32, precision=HI)
        s2c = jnp.dot(oht, z * z, preferred_element_type=F32, precision=HI)
        cgc = jnp.dot(oht, jnp.ones_like(z), preferred_element_type=F32, precision=HI)

        @pl.when(i == 0)
        def _():
            s1_ref[...] = jnp.zeros_like(s1_ref)
            s2_ref[...] = jnp.zeros_like(s2_ref)
            cg_ref[...] = jnp.zeros_like(cg_ref)
        s1_ref[...] += s1c
        s2_ref[...] += s2c
        cg_ref[...] += cgc

    full = lambda shape: pl.BlockSpec(shape, lambda i: tuple(0 for _ in shape))
    return pl.pallas_call(
        body,
        grid=(NBLK,),
        in_specs=[
            pl.BlockSpec((NB, 128), lambda i: (i, 0)),
            pl.BlockSpec((NB, 512), lambda i: (i, 0)),
            pl.BlockSpec((2, NB, 4), lambda i: (0, i, 0)),
            pl.BlockSpec((NB, 1), lambda i: (i, 0)),
            pl.BlockSpec((1, NB), lambda i: (0, i)),
            full((4, 512)),
            full((128, 256)),
            full((512, 256)),
            full((1, 256)),
        ],
        out_specs=[
            pl.BlockSpec((NB, 256), lambda i: (i, 0)),
            full((G, 256)),
            full((G, 256)),
            full((G, 256)),
        ],
        out_shape=[
            jax.ShapeDtypeStruct((NPAD, 256), F32),
            jax.ShapeDtypeStruct((G, 256), F32),
            jax.ShapeDtypeStruct((G, 256), F32),
            jax.ShapeDtypeStruct((G, 256), F32),
        ],
    )(h1, accr, scr, bcol, brow, expand, r2, w2s, b2r)


def _tc_final(z2, bcol, brow, a2, b2n, cg2, lw, lb):
    def body(z_ref, bc_ref, br_ref, a_ref, b_ref, cg_ref, lw_ref, lb_ref,
             out_ref, p_acc):
        i = pl.program_id(0)

        @pl.when(i == 0)
        def _():
            p_acc[...] = jnp.zeros_like(p_acc)
        oh = (bc_ref[...] == _iota_row(G)).astype(F32)
        an = jnp.dot(oh, a_ref[...], preferred_element_type=F32, precision=HI)
        bn = jnp.dot(oh, b_ref[...], preferred_element_type=F32, precision=HI)
        h = jnp.maximum(an * z_ref[...] + bn, 0.0)
        oht = (br_ref[...] == lax.broadcasted_iota(I32, (G, NB), 0).astype(F32)
               ).astype(F32)
        p_acc[...] += jnp.dot(oht, h, preferred_element_type=F32, precision=HI)

        @pl.when(i == NBLK - 1)
        def _():
            pooled = p_acc[...] / jnp.maximum(cg_ref[...], 1.0)
            out_ref[...] = (jnp.dot(pooled, lw_ref[...],
                                    preferred_element_type=F32, precision=HI)
                            + lb_ref[...])

    full = lambda shape: pl.BlockSpec(shape, lambda i: tuple(0 for _ in shape))
    return pl.pallas_call(
        body,
        grid=(NBLK,),
        in_specs=[
            pl.BlockSpec((NB, 256), lambda i: (i, 0)),
            pl.BlockSpec((NB, 1), lambda i: (i, 0)),
            pl.BlockSpec((1, NB), lambda i: (0, i)),
            full((G, 256)),
            full((G, 256)),
            full((G, 256)),
            full((256, 10)),
            full((1, 10)),
        ],
        out_specs=full((G, 10)),
        out_shape=jax.ShapeDtypeStruct((G, 10), F32),
        scratch_shapes=[pltpu.VMEM((G, 256), F32)],
    )(z2, bcol, brow, a2, b2n, cg2, lw, lb)


def kernel(x, edge_index, edge_type, batch, w1, root1, b1, w2, root2, b2,
           g1w, g1b, g1a, g2w, g2b, g2a, lin_w, lin_b):
    n = x.shape[0]
    e = edge_index.shape[1]

    src = edge_index[0].astype(I32)
    rid = (edge_index[1] * 4 + edge_type).astype(I32)
    src_p = jnp.concatenate([src, jnp.zeros((E_PAD - e,), I32)])
    rid_p = jnp.concatenate([rid, jnp.full((E_PAD - e,), 4 * n, I32)])

    xf = x[:, 0].astype(F32)
    xp = jnp.pad(x.astype(F32), ((0, NPAD - n), (0, 0)))
    bf = batch.astype(F32)
    bcol = jnp.pad(bf, (0, NPAD - n), constant_values=PAD_G)[:, None]
    brow = bcol.reshape(1, NPAD)

    w1m = w1[:, 0, :].astype(F32)
    r1 = root1.astype(F32)
    b1r = b1.reshape(1, -1).astype(F32)
    w2s = w2.reshape(4 * 128, 256).astype(F32)
    b2r = b2.reshape(1, -1).astype(F32)
    expand = jnp.kron(jnp.eye(4, dtype=F32), jnp.ones((1, 128), F32))

    outv, outc = _edge_scalar_sc(xf, src_p, rid_p)
    svr = outv.reshape(NC, NPAD, 4)
    scr = outc.reshape(NC, NPAD, 4)

    z1, s1, s2, cg = _tc_layer1(xp, svr, scr, bcol, brow, w1m, r1, b1r)
    a1, b1n = _tc_norm_params(s1, s2, cg, g1w.reshape(1, -1),
                              g1b.reshape(1, -1), g1a.reshape(1, -1), 128)
    h1 = _tc_apply_relu(z1, bcol, a1, b1n)

    acc = _edge_rows_sc(h1, src_p, rid_p)
    accr = acc.reshape(RTOT_E // 4, 512)

    z2, s1b, s2b, cg2 = _tc_layer2(h1, accr, scr, bcol, brow, expand,
                                   root2.astype(F32), w2s, b2r)
    a2, b2n = _tc_norm_params(s1b, s2b, cg2, g2w.reshape(1, -1),
                              g2b.reshape(1, -1), g2a.reshape(1, -1), 256)

    return _tc_final(z2, bcol, brow, a2, b2n, cg2,
                     lin_w.astype(F32), lin_b.reshape(1, -1).astype(F32))


# speedup vs baseline: 5.5956x; 1.0007x over previous
import functools

import jax
import jax.numpy as jnp
from jax import lax
from jax.experimental import pallas as pl
from jax.experimental.pallas import tpu as pltpu
from jax.experimental.pallas import tpu_sc as plsc

F32 = jnp.float32
HI = jax.lax.Precision.HIGHEST
I32 = jnp.int32

NC = 2
NS = 16
LANES = 16
NW = NC * NS

G = 64
PAD_G = 64.0

NB = 1024
NPAD = 50176
NBLK = NPAD // NB

E_PAD = 819200
EPW = E_PAD // NW
CK = 1600
KG = 64
DKG = 2 * KG
CB = CK + DKG + LANES

RID_CAP = 4 * NPAD
NP = 9
SCROWS = 12160
ACC_ROWS = 12288
TROWS = ACC_ROWS // NS
ZROWS = 64
CROWS = SCROWS // NS
RTOT_E = NP * NC * SCROWS


def _edge_scalar_sc(xf, src_p, rid_p):
    mesh = plsc.VectorSubcoreMesh(
        core_axis_name="c", subcore_axis_name="s", num_cores=NC,
        num_subcores=NS)
    zslice = RID_CAP // NS

    @functools.partial(
        pl.kernel,
        out_type=(jax.ShapeDtypeStruct((NC, RID_CAP), F32),
                  jax.ShapeDtypeStruct((NC, RID_CAP), F32)),
        mesh=mesh,
        scratch_types=[
            pltpu.VMEM((CK,), I32),
            pltpu.VMEM((CK,), I32),
            pltpu.VMEM((CK,), F32),
            pltpu.VMEM((CK,), F32),
            pltpu.VMEM_SHARED((RID_CAP,), F32),
            pltpu.VMEM_SHARED((RID_CAP,), F32),
            pltpu.SemaphoreType.DMA,
        ],
    )
    def k(x_hbm, src_hbm, rid_hbm, z_hbm, outv_hbm, outc_hbm,
          src_v, rid_v, val_v, ones_v, accv, accc, sem):
        c = lax.axis_index("c")
        s = lax.axis_index("s")
        wid = s * NC + c

        pltpu.sync_copy(z_hbm.at[pl.ds(s * zslice, zslice)],
                        accv.at[pl.ds(s * zslice, zslice)])
        pltpu.sync_copy(z_hbm.at[pl.ds(s * zslice, zslice)],
                        accc.at[pl.ds(s * zslice, zslice)])

        def fill(i, _):
            ones_v[pl.ds(i * LANES, LANES)] = jnp.full((LANES,), 1.0, F32)
            return 0
        lax.fori_loop(0, CK // LANES, fill, 0)
        plsc.subcore_barrier()

        def step(i, _):
            base = wid * EPW + i * CK
            pltpu.sync_copy(src_hbm.at[pl.ds(base, CK)], src_v)
            pltpu.sync_copy(rid_hbm.at[pl.ds(base, CK)], rid_v)
            pltpu.async_copy(x_hbm.at[src_v], val_v, sem).wait()
            pltpu.sync_copy(val_v, accv.at[rid_v], add=True)
            pltpu.sync_copy(ones_v, accc.at[rid_v], add=True)
            return 0
        lax.fori_loop(0, EPW // CK, step, 0)
        plsc.subcore_barrier()

        pltpu.sync_copy(accv.at[pl.ds(s * zslice, zslice)],
                        outv_hbm.at[c, pl.ds(s * zslice, zslice)])
        pltpu.sync_copy(accc.at[pl.ds(s * zslice, zslice)],
                        outc_hbm.at[c, pl.ds(s * zslice, zslice)])

    zeros = jnp.zeros((RID_CAP,), F32)
    return k(xf, src_p, rid_p, zeros)


def _edge_rows_sc(h1, src_p, rid_p):
    mesh = plsc.VectorSubcoreMesh(
        core_axis_name="c", subcore_axis_name="s", num_cores=NC,
        num_subcores=NS)

    @functools.partial(
        pl.kernel,
        out_type=jax.ShapeDtypeStruct((RTOT_E, 128), F32),
        mesh=mesh,
        scratch_types=[
            pltpu.VMEM((CB,), I32),
            pltpu.VMEM((CB,), I32),
            pltpu.VMEM((CK,), I32),
            pltpu.VMEM((CK,), I32),
            pltpu.VMEM((KG,), I32),
            pltpu.VMEM((KG,), I32),
            pltpu.VMEM((KG,), I32),
            pltpu.VMEM((KG,), I32),
            pltpu.VMEM((KG, 128), F32),
            pltpu.VMEM((KG, 128), F32),
            pltpu.VMEM((ZROWS, 128), F32),
            pltpu.VMEM_SHARED((ACC_ROWS, 128), F32),
            pltpu.SemaphoreType.DMA,
            pltpu.SemaphoreType.DMA,
        ],
        compiler_params=pltpu.CompilerParams(needs_layout_passes=False),
    )
    def k(h_hbm, src_hbm, rid_hbm, out_hbm,
          src_c, lid_c, rid_v, src_v, idxg0, idxs0, idxg1, idxs1,
          rows0, rows1, zrows, acc, sem0, sem1):
        c = lax.axis_index("c")
        s = lax.axis_index("s")
        ept = E_PAD // NS

        def zfill(i, _):
            for kk in range(128 // LANES):
                zrows[i, pl.ds(kk * LANES, LANES)] = (
                    jnp.zeros((LANES,), F32))
            return 0
        lax.fori_loop(0, ZROWS, zfill, 0)
        iot = lax.iota(I32, LANES)

        def drain(t, _):
            def cp(j, _):
                idxg0[pl.ds(j * LANES, LANES)] = (
                    src_c[pl.ds(t * DKG + j * LANES, LANES)])
                idxs0[pl.ds(j * LANES, LANES)] = (
                    lid_c[pl.ds(t * DKG + j * LANES, LANES)])
                idxg1[pl.ds(j * LANES, LANES)] = (
                    src_c[pl.ds(t * DKG + KG + j * LANES, LANES)])
                idxs1[pl.ds(j * LANES, LANES)] = (
                    lid_c[pl.ds(t * DKG + KG + j * LANES, LANES)])
                return 0
            lax.fori_loop(0, KG // LANES, cp, 0)
            d0 = pltpu.async_copy(h_hbm.at[idxg0], rows0, sem0)
            d1 = pltpu.async_copy(h_hbm.at[idxg1], rows1, sem1)
            d0.wait()
            pltpu.sync_copy(rows0, acc.at[idxs0], add=True)
            d1.wait()
            pltpu.sync_copy(rows1, acc.at[idxs1], add=True)
            return 0

        def pass_body(p, _):
            r0 = (p * NC + c) * SCROWS

            def zero_acc(kk, _):
                pltpu.sync_copy(
                    zrows, acc.at[pl.ds(s * TROWS + kk * ZROWS, ZROWS)])
                return 0
            lax.fori_loop(0, TROWS // ZROWS, zero_acc, 0)
            plsc.subcore_barrier()

            def chunk(i, off):
                base = s * ept + i * CK
                pltpu.sync_copy(src_hbm.at[pl.ds(base, CK)], src_v)
                pltpu.sync_copy(rid_hbm.at[pl.ds(base, CK)], rid_v)

                def vec(j, off):
                    rv = rid_v[pl.ds(j * LANES, LANES)]
                    sv = src_v[pl.ds(j * LANES, LANES)]
                    lid = rv - r0
                    m = (lid >= 0) & (lid < SCROWS)
                    cs = plsc.cumsum(jnp.where(m, 1.0, 0.0))
                    pos = off + cs.astype(I32) - 1
                    plsc.store_scatter(lid_c, [pos], lid, mask=m)
                    plsc.store_scatter(src_c, [pos], sv, mask=m)
                    return off + cs[LANES - 1].astype(I32)
                off = lax.fori_loop(0, CK // LANES, vec, off)

                nfull = off // DKG
                lax.fori_loop(0, nfull, drain, 0)
                rem = off - nfull * DKG

                def mv(kk, _):
                    lid_c[pl.ds(kk * LANES, LANES)] = (
                        lid_c[pl.ds(nfull * DKG + kk * LANES, LANES)])
                    src_c[pl.ds(kk * LANES, LANES)] = (
                        src_c[pl.ds(nfull * DKG + kk * LANES, LANES)])
                    return 0
                lax.fori_loop(0, (rem + LANES - 1) // LANES, mv, 0)
                return rem
            off = lax.fori_loop(0, (E_PAD // NS) // CK, chunk,
                                jnp.zeros((), I32))

            for kk in range(DKG // LANES):
                pos = off + kk * LANES + iot
                plsc.store_scatter(
                    lid_c, [pos], jnp.full((LANES,), SCROWS, I32))
                plsc.store_scatter(
                    src_c, [pos], jnp.zeros((LANES,), I32))
            lax.fori_loop(0, (off + DKG - 1) // DKG, drain, 0)
            plsc.subcore_barrier()

            pltpu.sync_copy(acc.at[pl.ds(s * CROWS, CROWS)],
                            out_hbm.at[pl.ds(r0 + s * CROWS, CROWS)])
            plsc.subcore_barrier()
            return 0
        lax.fori_loop(0, NP, pass_body, 0)

    return k(h1, src_p, rid_p)


def _iota_row(width):
    return lax.broadcasted_iota(I32, (1, width), 1).astype(F32)


def _tc_layer1(xp, svr, scr, bcol, brow, w1m, r1, b1r):
    def body(x_ref, sv_ref, sc_ref, bc_ref, br_ref, w_ref, r_ref, b_ref,
             z_ref, s1_ref, s2_ref, cg_ref):
        i = pl.program_id(0)
        xb = x_ref[...]
        sv = sv_ref[0] + sv_ref[1]
        scnt = sc_ref[0] + sc_ref[1]
        mean = sv / jnp.maximum(scnt, 1.0)
        z = (xb * r_ref[...]
             + jnp.dot(mean, w_ref[...], preferred_element_type=F32, precision=HI)
             + b_ref[...])
        z_ref[...] = z
        oht = (br_ref[...] == lax.broadcasted_iota(I32, (G, NB), 0).astype(F32)
               ).astype(F32)
        s1c = jnp.dot(oht, z, preferred_element_type=F32, precision=HI)
        s2c = jnp.dot(oht, z * z, preferred_element_type=F32, precision=HI)
        cgc = jnp.dot(oht, jnp.ones_like(z), preferred_element_type=F32, precision=HI)

        @pl.when(i == 0)
        def _():
            s1_ref[...] = jnp.zeros_like(s1_ref)
            s2_ref[...] = jnp.zeros_like(s2_ref)
            cg_ref[...] = jnp.zeros_like(cg_ref)
        s1_ref[...] += s1c
        s2_ref[...] += s2c
        cg_ref[...] += cgc

    full = lambda shape: pl.BlockSpec(shape, lambda i: tuple(0 for _ in shape))
    return pl.pallas_call(
        body,
        grid=(NBLK,),
        in_specs=[
            pl.BlockSpec((NB, 1), lambda i: (i, 0)),
            pl.BlockSpec((2, NB, 4), lambda i: (0, i, 0)),
            pl.BlockSpec((2, NB, 4), lambda i: (0, i, 0)),
            pl.BlockSpec((NB, 1), lambda i: (i, 0)),
            pl.BlockSpec((1, NB), lambda i: (0, i)),
            full((4, 128)),
            full((1, 128)),
            full((1, 128)),
        ],
        out_specs=[
            pl.BlockSpec((NB, 128), lambda i: (i, 0)),
            full((G, 128)),
            full((G, 128)),
            full((G, 128)),
        ],
        out_shape=[
            jax.ShapeDtypeStruct((NPAD, 128), F32),
            jax.ShapeDtypeStruct((G, 128), F32),
            jax.ShapeDtypeStruct((G, 128), F32),
            jax.ShapeDtypeStruct((G, 128), F32),
        ],
    )(xp, svr, scr, bcol, brow, w1m, r1, b1r)


def _tc_norm_params(s1, s2, cg, gw, gb, ga, width):
    def body(s1_ref, s2_ref, cg_ref, gw_ref, gb_ref, ga_ref, a_ref, b_ref):
        c = jnp.maximum(cg_ref[...], 1.0)
        m = s1_ref[...] / c
        a = ga_ref[...]
        var = s2_ref[...] / c - (2.0 * a - a * a) * m * m
        aa = gw_ref[...] * lax.rsqrt(var + 1e-5)
        a_ref[...] = aa
        b_ref[...] = gb_ref[...] - aa * a * m

    full = lambda shape: pl.BlockSpec(shape, lambda: tuple(0 for _ in shape))
    return pl.pallas_call(
        body,
        in_specs=[full((G, width))] * 3 + [full((1, width))] * 3,
        out_specs=[full((G, width))] * 2,
        out_shape=[jax.ShapeDtypeStruct((G, width), F32)] * 2,
    )(s1, s2, cg, gw, gb, ga)


def _tc_apply_relu(z1, bcol, a1, b1n):
    def body(z_ref, bc_ref, a_ref, b_ref, h_ref):
        oh = (bc_ref[...] == _iota_row(G)).astype(F32)
        an = jnp.dot(oh, a_ref[...], preferred_element_type=F32, precision=HI)
        bn = jnp.dot(oh, b_ref[...], preferred_element_type=F32, precision=HI)
        h_ref[...] = jnp.maximum(an * z_ref[...] + bn, 0.0)

    full = lambda shape: pl.BlockSpec(shape, lambda i: tuple(0 for _ in shape))
    return pl.pallas_call(
        body,
        grid=(NBLK,),
        in_specs=[
            pl.BlockSpec((NB, 128), lambda i: (i, 0)),
            pl.BlockSpec((NB, 1), lambda i: (i, 0)),
            full((G, 128)),
            full((G, 128)),
        ],
        out_specs=pl.BlockSpec((NB, 128), lambda i: (i, 0)),
        out_shape=jax.ShapeDtypeStruct((NPAD, 128), F32),
    )(z1, bcol, a1, b1n)


def _tc_layer2(h1, accr, scr, bcol, brow, expand, r2, w2s, b2r):
    def body(h_ref, acc_ref, sc_ref, bc_ref, br_ref, e_ref, r_ref, w_ref,
             b_ref, z_ref, s1_ref, s2_ref, cg_ref):
        i = pl.program_id(0)
        cnt = sc_ref[0] + sc_ref[1]
        recip = 1.0 / jnp.maximum(cnt, 1.0)
        rb = jnp.dot(recip, e_ref[...], preferred_element_type=F32, precision=HI)
        mean = acc_ref[...] * rb
        z = (jnp.dot(h_ref[...], r_ref[...], preferred_element_type=F32, precision=HI)
             + jnp.dot(mean, w_ref[...], preferred_element_type=F32, precision=HI)
             + b_ref[...])
        z_ref[...] = z
        oht = (br_ref[...] == lax.broadcasted_iota(I32, (G, NB), 0).astype(F32)
               ).astype(F32)
        s1c = jnp.dot(oht, z, preferred_element_type=F32, precision=HI)
        s2c = jnp.dot(oht, z * z, preferred_element_type=F32, precision=HI)
        cgc = jnp.dot(oht, jnp.ones_like(z), preferred_element_type=F32, precision=HI)

        @pl.when(i == 0)
        def _():
            s1_ref[...] = jnp.zeros_like(s1_ref)
            s2_ref[...] = jnp.zeros_like(s2_ref)
            cg_ref[...] = jnp.zeros_like(cg_ref)
        s1_ref[...] += s1c
        s2_ref[...] += s2c
        cg_ref[...] += cgc

    full = lambda shape: pl.BlockSpec(shape, lambda i: tuple(0 for _ in shape))
    return pl.pallas_call(
        body,
        grid=(NBLK,),
        in_specs=[
            pl.BlockSpec((NB, 128), lambda i: (i, 0)),
            pl.BlockSpec((NB, 512), lambda i: (i, 0)),
            pl.BlockSpec((2, NB, 4), lambda i: (0, i, 0)),
            pl.BlockSpec((NB, 1), lambda i: (i, 0)),
            pl.BlockSpec((1, NB), lambda i: (0, i)),
            full((4, 512)),
            full((128, 256)),
            full((512, 256)),
            full((1, 256)),
        ],
        out_specs=[
            pl.BlockSpec((NB, 256), lambda i: (i, 0)),
            full((G, 256)),
            full((G, 256)),
            full((G, 256)),
        ],
        out_shape=[
            jax.ShapeDtypeStruct((NPAD, 256), F32),
            jax.ShapeDtypeStruct((G, 256), F32),
            jax.ShapeDtypeStruct((G, 256), F32),
            jax.ShapeDtypeStruct((G, 256), F32),
        ],
    )(h1, accr, scr, bcol, brow, expand, r2, w2s, b2r)


def _tc_final(z2, bcol, brow, a2, b2n, cg2, lw, lb):
    def body(z_ref, bc_ref, br_ref, a_ref, b_ref, cg_ref, lw_ref, lb_ref,
             out_ref, p_acc):
        i = pl.program_id(0)

        @pl.when(i == 0)
        def _():
            p_acc[...] = jnp.zeros_like(p_acc)
        oh = (bc_ref[...] == _iota_row(G)).astype(F32)
        an = jnp.dot(oh, a_ref[...], preferred_element_type=F32, precision=HI)
        bn = jnp.dot(oh, b_ref[...], preferred_element_type=F32, precision=HI)
        h = jnp.maximum(an * z_ref[...] + bn, 0.0)
        oht = (br_ref[...] == lax.broadcasted_iota(I32, (G, NB), 0).astype(F32)
               ).astype(F32)
        p_acc[...] += jnp.dot(oht, h, preferred_element_type=F32, precision=HI)

        @pl.when(i == NBLK - 1)
        def _():
            pooled = p_acc[...] / jnp.maximum(cg_ref[...], 1.0)
            out_ref[...] = (jnp.dot(pooled, lw_ref[...],
                                    preferred_element_type=F32, precision=HI)
                            + lb_ref[...])

    full = lambda shape: pl.BlockSpec(shape, lambda i: tuple(0 for _ in shape))
    return pl.pallas_call(
        body,
        grid=(NBLK,),
        in_specs=[
            pl.BlockSpec((NB, 256), lambda i: (i, 0)),
            pl.BlockSpec((NB, 1), lambda i: (i, 0)),
            pl.BlockSpec((1, NB), lambda i: (0, i)),
            full((G, 256)),
            full((G, 256)),
            full((G, 256)),
            full((256, 10)),
            full((1, 10)),
        ],
        out_specs=full((G, 10)),
        out_shape=jax.ShapeDtypeStruct((G, 10), F32),
        scratch_shapes=[pltpu.VMEM((G, 256), F32)],
    )(z2, bcol, brow, a2, b2n, cg2, lw, lb)


def kernel(x, edge_index, edge_type, batch, w1, root1, b1, w2, root2, b2,
           g1w, g1b, g1a, g2w, g2b, g2a, lin_w, lin_b):
    n = x.shape[0]
    e = edge_index.shape[1]

    src = edge_index[0].astype(I32)
    rid = (edge_index[1] * 4 + edge_type).astype(I32)
    src_p = jnp.concatenate([src, jnp.zeros((E_PAD - e,), I32)])
    rid_p = jnp.concatenate([rid, jnp.full((E_PAD - e,), 4 * n, I32)])

    xf = x[:, 0].astype(F32)
    xp = jnp.pad(x.astype(F32), ((0, NPAD - n), (0, 0)))
    bf = batch.astype(F32)
    bcol = jnp.pad(bf, (0, NPAD - n), constant_values=PAD_G)[:, None]
    brow = bcol.reshape(1, NPAD)

    w1m = w1[:, 0, :].astype(F32)
    r1 = root1.astype(F32)
    b1r = b1.reshape(1, -1).astype(F32)
    w2s = w2.reshape(4 * 128, 256).astype(F32)
    b2r = b2.reshape(1, -1).astype(F32)
    expand = jnp.kron(jnp.eye(4, dtype=F32), jnp.ones((1, 128), F32))

    outv, outc = _edge_scalar_sc(xf, src_p, rid_p)
    svr = outv.reshape(NC, NPAD, 4)
    scr = outc.reshape(NC, NPAD, 4)

    z1, s1, s2, cg = _tc_layer1(xp, svr, scr, bcol, brow, w1m, r1, b1r)
    a1, b1n = _tc_norm_params(s1, s2, cg, g1w.reshape(1, -1),
                              g1b.reshape(1, -1), g1a.reshape(1, -1), 128)
    h1 = _tc_apply_relu(z1, bcol, a1, b1n)

    acc = _edge_rows_sc(h1, src_p, rid_p)
    accr = acc.reshape(RTOT_E // 4, 512)

    z2, s1b, s2b, cg2 = _tc_layer2(h1, accr, scr, bcol, brow, expand,
                                   root2.astype(F32), w2s, b2r)
    a2, b2n = _tc_norm_params(s1b, s2b, cg2, g2w.reshape(1, -1),
                              g2b.reshape(1, -1), g2a.reshape(1, -1), 256)

    return _tc_final(z2, bcol, brow, a2, b2n, cg2,
                     lin_w.astype(F32), lin_b.reshape(1, -1).astype(F32))


# speedup vs baseline: 5.8743x; 1.0498x over previous
import functools

import jax
import jax.numpy as jnp
from jax import lax
from jax.experimental import pallas as pl
from jax.experimental.pallas import tpu as pltpu
from jax.experimental.pallas import tpu_sc as plsc

F32 = jnp.float32
HI = jax.lax.Precision.HIGHEST
I32 = jnp.int32

NC = 2
NS = 16
LANES = 16
NW = NC * NS

G = 64
PAD_G = 64.0

NB = 1024
NPAD = 50176
NBLK = NPAD // NB

E_PAD = 819200
EPW = E_PAD // NW
CK = 1600
KG = 64
DKG = 2 * KG
CB = CK + DKG + LANES

RID_CAP = 4 * NPAD
NP = 9
SCROWS = 12160
ACC_ROWS = 12288
TROWS = ACC_ROWS // NS
ZROWS = 64
CROWS = SCROWS // NS
RTOT_E = NP * NC * SCROWS


def _edge_scalar_sc(xf, src_p, rid_p):
    mesh = plsc.VectorSubcoreMesh(
        core_axis_name="c", subcore_axis_name="s", num_cores=NC,
        num_subcores=NS)
    zslice = RID_CAP // NS

    @functools.partial(
        pl.kernel,
        out_type=(jax.ShapeDtypeStruct((NC, RID_CAP), F32),
                  jax.ShapeDtypeStruct((NC, RID_CAP), F32)),
        mesh=mesh,
        scratch_types=[
            pltpu.VMEM((CK,), I32),
            pltpu.VMEM((CK,), I32),
            pltpu.VMEM((CK,), F32),
            pltpu.VMEM((CK,), F32),
            pltpu.VMEM_SHARED((RID_CAP,), F32),
            pltpu.VMEM_SHARED((RID_CAP,), F32),
            pltpu.SemaphoreType.DMA,
        ],
    )
    def k(x_hbm, src_hbm, rid_hbm, z_hbm, outv_hbm, outc_hbm,
          src_v, rid_v, val_v, ones_v, accv, accc, sem):
        c = lax.axis_index("c")
        s = lax.axis_index("s")
        wid = s * NC + c

        pltpu.sync_copy(z_hbm.at[pl.ds(s * zslice, zslice)],
                        accv.at[pl.ds(s * zslice, zslice)])
        pltpu.sync_copy(z_hbm.at[pl.ds(s * zslice, zslice)],
                        accc.at[pl.ds(s * zslice, zslice)])

        def fill(i, _):
            ones_v[pl.ds(i * LANES, LANES)] = jnp.full((LANES,), 1.0, F32)
            return 0
        lax.fori_loop(0, CK // LANES, fill, 0)
        plsc.subcore_barrier()

        def step(i, _):
            base = wid * EPW + i * CK
            pltpu.sync_copy(src_hbm.at[pl.ds(base, CK)], src_v)
            pltpu.sync_copy(rid_hbm.at[pl.ds(base, CK)], rid_v)
            pltpu.async_copy(x_hbm.at[src_v], val_v, sem).wait()
            pltpu.sync_copy(val_v, accv.at[rid_v], add=True)
            pltpu.sync_copy(ones_v, accc.at[rid_v], add=True)
            return 0
        lax.fori_loop(0, EPW // CK, step, 0)
        plsc.subcore_barrier()

        pltpu.sync_copy(accv.at[pl.ds(s * zslice, zslice)],
                        outv_hbm.at[c, pl.ds(s * zslice, zslice)])
        pltpu.sync_copy(accc.at[pl.ds(s * zslice, zslice)],
                        outc_hbm.at[c, pl.ds(s * zslice, zslice)])

    zeros = jnp.zeros((RID_CAP,), F32)
    return k(xf, src_p, rid_p, zeros)


def _edge_rows_sc(h1, src_p, rid_p):
    mesh = plsc.VectorSubcoreMesh(
        core_axis_name="c", subcore_axis_name="s", num_cores=NC,
        num_subcores=NS)

    @functools.partial(
        pl.kernel,
        out_type=jax.ShapeDtypeStruct((RTOT_E, 128), F32),
        mesh=mesh,
        scratch_types=[
            pltpu.VMEM((CB,), I32),
            pltpu.VMEM((CB,), I32),
            pltpu.VMEM((CK,), I32),
            pltpu.VMEM((CK,), I32),
            pltpu.VMEM((KG,), I32),
            pltpu.VMEM((KG,), I32),
            pltpu.VMEM((KG,), I32),
            pltpu.VMEM((KG,), I32),
            pltpu.VMEM((KG, 128), F32),
            pltpu.VMEM((KG, 128), F32),
            pltpu.VMEM((ZROWS, 128), F32),
            pltpu.VMEM_SHARED((ACC_ROWS, 128), F32),
            pltpu.SemaphoreType.DMA,
            pltpu.SemaphoreType.DMA,
        ],
        compiler_params=pltpu.CompilerParams(needs_layout_passes=False),
    )
    def k(h_hbm, src_hbm, rid_hbm, out_hbm,
          src_c, lid_c, rid_v, src_v, idxg0, idxs0, idxg1, idxs1,
          rows0, rows1, zrows, acc, sem0, sem1):
        c = lax.axis_index("c")
        s = lax.axis_index("s")
        ept = E_PAD // NS

        def zfill(i, _):
            for kk in range(128 // LANES):
                zrows[i, pl.ds(kk * LANES, LANES)] = (
                    jnp.zeros((LANES,), F32))
            return 0
        lax.fori_loop(0, ZROWS, zfill, 0)
        iot = lax.iota(I32, LANES)

        def drain(t, _):
            def cp(j, _):
                idxg0[pl.ds(j * LANES, LANES)] = (
                    src_c[pl.ds(t * DKG + j * LANES, LANES)])
                idxs0[pl.ds(j * LANES, LANES)] = (
                    lid_c[pl.ds(t * DKG + j * LANES, LANES)])
                idxg1[pl.ds(j * LANES, LANES)] = (
                    src_c[pl.ds(t * DKG + KG + j * LANES, LANES)])
                idxs1[pl.ds(j * LANES, LANES)] = (
                    lid_c[pl.ds(t * DKG + KG + j * LANES, LANES)])
                return 0
            lax.fori_loop(0, KG // LANES, cp, 0)
            d0 = pltpu.async_copy(h_hbm.at[idxg0], rows0, sem0)
            d1 = pltpu.async_copy(h_hbm.at[idxg1], rows1, sem1)
            d0.wait()
            pltpu.sync_copy(rows0, acc.at[idxs0], add=True)
            d1.wait()
            pltpu.sync_copy(rows1, acc.at[idxs1], add=True)
            return 0

        def pass_body(p, _):
            r0 = (p * NC + c) * SCROWS

            def zero_acc(kk, _):
                pltpu.sync_copy(
                    zrows, acc.at[pl.ds(s * TROWS + kk * ZROWS, ZROWS)])
                return 0
            lax.fori_loop(0, TROWS // ZROWS, zero_acc, 0)
            plsc.subcore_barrier()

            def chunk(i, off):
                base = s * ept + i * CK
                pltpu.sync_copy(src_hbm.at[pl.ds(base, CK)], src_v)
                pltpu.sync_copy(rid_hbm.at[pl.ds(base, CK)], rid_v)

                def vec(j, off):
                    rv = rid_v[pl.ds(j * LANES, LANES)]
                    sv = src_v[pl.ds(j * LANES, LANES)]
                    lid = rv - r0
                    m = (lid >= 0) & (lid < SCROWS)
                    cs = plsc.cumsum(jnp.where(m, 1.0, 0.0))
                    pos = off + cs.astype(I32) - 1
                    plsc.store_scatter(lid_c, [pos], lid, mask=m)
                    plsc.store_scatter(src_c, [pos], sv, mask=m)
                    return off + cs[LANES - 1].astype(I32)
                off = lax.fori_loop(0, CK // LANES, vec, off)

                nfull = off // DKG
                lax.fori_loop(0, nfull, drain, 0)
                rem = off - nfull * DKG

                def mv(kk, _):
                    lid_c[pl.ds(kk * LANES, LANES)] = (
                        lid_c[pl.ds(nfull * DKG + kk * LANES, LANES)])
                    src_c[pl.ds(kk * LANES, LANES)] = (
                        src_c[pl.ds(nfull * DKG + kk * LANES, LANES)])
                    return 0
                lax.fori_loop(0, (rem + LANES - 1) // LANES, mv, 0)
                return rem
            off = lax.fori_loop(0, (E_PAD // NS) // CK, chunk,
                                jnp.zeros((), I32))

            for kk in range(DKG // LANES):
                pos = off + kk * LANES + iot
                plsc.store_scatter(
                    lid_c, [pos], jnp.full((LANES,), SCROWS, I32))
                plsc.store_scatter(
                    src_c, [pos], jnp.zeros((LANES,), I32))
            lax.fori_loop(0, (off + DKG - 1) // DKG, drain, 0)
            plsc.subcore_barrier()

            pltpu.sync_copy(acc.at[pl.ds(s * CROWS, CROWS)],
                            out_hbm.at[pl.ds(r0 + s * CROWS, CROWS)])
            plsc.subcore_barrier()
            return 0
        lax.fori_loop(0, NP, pass_body, 0)

    return k(h1, src_p, rid_p)


def _iota_row(width):
    return lax.broadcasted_iota(I32, (1, width), 1).astype(F32)


def _tc_layer1(xp, svr, scr, bcol, brow, w1m, r1, b1r):
    def body(x_ref, sv_ref, sc_ref, bc_ref, br_ref, w_ref, r_ref, b_ref,
             z_ref, s1_ref, s2_ref, cg_ref):
        i = pl.program_id(0)
        xb = x_ref[...]
        sv = sv_ref[0] + sv_ref[1]
        scnt = sc_ref[0] + sc_ref[1]
        mean = sv / jnp.maximum(scnt, 1.0)
        z = (xb * r_ref[...]
             + jnp.dot(mean, w_ref[...], preferred_element_type=F32, precision=HI)
             + b_ref[...])
        z_ref[...] = z
        oht = (br_ref[...] == lax.broadcasted_iota(I32, (G, NB), 0).astype(F32)
               ).astype(F32)
        s1c = jnp.dot(oht, z, preferred_element_type=F32, precision=HI)
        s2c = jnp.dot(oht, z * z, preferred_element_type=F32, precision=HI)
        cgc = jnp.dot(oht, jnp.ones_like(z), preferred_element_type=F32, precision=HI)

        @pl.when(i == 0)
        def _():
            s1_ref[...] = jnp.zeros_like(s1_ref)
            s2_ref[...] = jnp.zeros_like(s2_ref)
            cg_ref[...] = jnp.zeros_like(cg_ref)
        s1_ref[...] += s1c
        s2_ref[...] += s2c
        cg_ref[...] += cgc

    full = lambda shape: pl.BlockSpec(shape, lambda i: tuple(0 for _ in shape))
    return pl.pallas_call(
        body,
        grid=(NBLK,),
        in_specs=[
            pl.BlockSpec((NB, 1), lambda i: (i, 0)),
            pl.BlockSpec((2, NB, 4), lambda i: (0, i, 0)),
            pl.BlockSpec((2, NB, 4), lambda i: (0, i, 0)),
            pl.BlockSpec((NB, 1), lambda i: (i, 0)),
            pl.BlockSpec((1, NB), lambda i: (0, i)),
            full((4, 128)),
            full((1, 128)),
            full((1, 128)),
        ],
        out_specs=[
            pl.BlockSpec((NB, 128), lambda i: (i, 0)),
            full((G, 128)),
            full((G, 128)),
            full((G, 128)),
        ],
        out_shape=[
            jax.ShapeDtypeStruct((NPAD, 128), F32),
            jax.ShapeDtypeStruct((G, 128), F32),
            jax.ShapeDtypeStruct((G, 128), F32),
            jax.ShapeDtypeStruct((G, 128), F32),
        ],
    )(xp, svr, scr, bcol, brow, w1m, r1, b1r)


def _tc_norm_params(s1, s2, cg, gw, gb, ga, width):
    def body(s1_ref, s2_ref, cg_ref, gw_ref, gb_ref, ga_ref, a_ref, b_ref):
        c = jnp.maximum(cg_ref[...], 1.0)
        m = s1_ref[...] / c
        a = ga_ref[...]
        var = s2_ref[...] / c - (2.0 * a - a * a) * m * m
        aa = gw_ref[...] * lax.rsqrt(var + 1e-5)
        a_ref[...] = aa
        b_ref[...] = gb_ref[...] - aa * a * m

    full = lambda shape: pl.BlockSpec(shape, lambda: tuple(0 for _ in shape))
    return pl.pallas_call(
        body,
        in_specs=[full((G, width))] * 3 + [full((1, width))] * 3,
        out_specs=[full((G, width))] * 2,
        out_shape=[jax.ShapeDtypeStruct((G, width), F32)] * 2,
    )(s1, s2, cg, gw, gb, ga)


def _tc_apply_relu(z1, bcol, a1, b1n):
    def body(z_ref, bc_ref, a_ref, b_ref, h_ref):
        oh = (bc_ref[...] == _iota_row(G)).astype(F32)
        an = jnp.dot(oh, a_ref[...], preferred_element_type=F32, precision=HI)
        bn = jnp.dot(oh, b_ref[...], preferred_element_type=F32, precision=HI)
        h_ref[...] = jnp.maximum(an * z_ref[...] + bn, 0.0)

    full = lambda shape: pl.BlockSpec(shape, lambda i: tuple(0 for _ in shape))
    return pl.pallas_call(
        body,
        grid=(NBLK,),
        in_specs=[
            pl.BlockSpec((NB, 128), lambda i: (i, 0)),
            pl.BlockSpec((NB, 1), lambda i: (i, 0)),
            full((G, 128)),
            full((G, 128)),
        ],
        out_specs=pl.BlockSpec((NB, 128), lambda i: (i, 0)),
        out_shape=jax.ShapeDtypeStruct((NPAD, 128), F32),
    )(z1, bcol, a1, b1n)


def _tc_layer2(h1, accr, scr, bcol, brow, expand, r2, w2s, b2r):
    def body(h_ref, acc_ref, sc_ref, bc_ref, br_ref, e_ref, r_ref, w_ref,
             b_ref, z_ref, s1_ref, s2_ref, cg_ref):
        i = pl.program_id(0)
        cnt = sc_ref[0] + sc_ref[1]
        recip = 1.0 / jnp.maximum(cnt, 1.0)
        rb = jnp.dot(recip, e_ref[...], preferred_element_type=F32, precision=HI)
        mean = acc_ref[...] * rb
        z = (jnp.dot(h_ref[...], r_ref[...], preferred_element_type=F32)
             + jnp.dot(mean, w_ref[...], preferred_element_type=F32)
             + b_ref[...])
        z_ref[...] = z
        oht = (br_ref[...] == lax.broadcasted_iota(I32, (G, NB), 0).astype(F32)
               ).astype(F32)
        s1c = jnp.dot(oht, z, preferred_element_type=F32, precision=HI)
        s2c = jnp.dot(oht, z * z, preferred_element_type=F32, precision=HI)
        cgc = jnp.dot(oht, jnp.ones_like(z), preferred_element_type=F32, precision=HI)

        @pl.when(i == 0)
        def _():
            s1_ref[...] = jnp.zeros_like(s1_ref)
            s2_ref[...] = jnp.zeros_like(s2_ref)
            cg_ref[...] = jnp.zeros_like(cg_ref)
        s1_ref[...] += s1c
        s2_ref[...] += s2c
        cg_ref[...] += cgc

    full = lambda shape: pl.BlockSpec(shape, lambda i: tuple(0 for _ in shape))
    return pl.pallas_call(
        body,
        grid=(NBLK,),
        in_specs=[
            pl.BlockSpec((NB, 128), lambda i: (i, 0)),
            pl.BlockSpec((NB, 512), lambda i: (i, 0)),
            pl.BlockSpec((2, NB, 4), lambda i: (0, i, 0)),
            pl.BlockSpec((NB, 1), lambda i: (i, 0)),
            pl.BlockSpec((1, NB), lambda i: (0, i)),
            full((4, 512)),
            full((128, 256)),
            full((512, 256)),
            full((1, 256)),
        ],
        out_specs=[
            pl.BlockSpec((NB, 256), lambda i: (i, 0)),
            full((G, 256)),
            full((G, 256)),
            full((G, 256)),
        ],
        out_shape=[
            jax.ShapeDtypeStruct((NPAD, 256), F32),
            jax.ShapeDtypeStruct((G, 256), F32),
            jax.ShapeDtypeStruct((G, 256), F32),
            jax.ShapeDtypeStruct((G, 256), F32),
        ],
    )(h1, accr, scr, bcol, brow, expand, r2, w2s, b2r)


def _tc_final(z2, bcol, brow, a2, b2n, cg2, lw, lb):
    def body(z_ref, bc_ref, br_ref, a_ref, b_ref, cg_ref, lw_ref, lb_ref,
             out_ref, p_acc):
        i = pl.program_id(0)

        @pl.when(i == 0)
        def _():
            p_acc[...] = jnp.zeros_like(p_acc)
        oh = (bc_ref[...] == _iota_row(G)).astype(F32)
        an = jnp.dot(oh, a_ref[...], preferred_element_type=F32, precision=HI)
        bn = jnp.dot(oh, b_ref[...], preferred_element_type=F32, precision=HI)
        h = jnp.maximum(an * z_ref[...] + bn, 0.0)
        oht = (br_ref[...] == lax.broadcasted_iota(I32, (G, NB), 0).astype(F32)
               ).astype(F32)
        p_acc[...] += jnp.dot(oht, h, preferred_element_type=F32, precision=HI)

        @pl.when(i == NBLK - 1)
        def _():
            pooled = p_acc[...] / jnp.maximum(cg_ref[...], 1.0)
            out_ref[...] = (jnp.dot(pooled, lw_ref[...],
                                    preferred_element_type=F32, precision=HI)
                            + lb_ref[...])

    full = lambda shape: pl.BlockSpec(shape, lambda i: tuple(0 for _ in shape))
    return pl.pallas_call(
        body,
        grid=(NBLK,),
        in_specs=[
            pl.BlockSpec((NB, 256), lambda i: (i, 0)),
            pl.BlockSpec((NB, 1), lambda i: (i, 0)),
            pl.BlockSpec((1, NB), lambda i: (0, i)),
            full((G, 256)),
            full((G, 256)),
            full((G, 256)),
            full((256, 10)),
            full((1, 10)),
        ],
        out_specs=full((G, 10)),
        out_shape=jax.ShapeDtypeStruct((G, 10), F32),
        scratch_shapes=[pltpu.VMEM((G, 256), F32)],
    )(z2, bcol, brow, a2, b2n, cg2, lw, lb)


def kernel(x, edge_index, edge_type, batch, w1, root1, b1, w2, root2, b2,
           g1w, g1b, g1a, g2w, g2b, g2a, lin_w, lin_b):
    n = x.shape[0]
    e = edge_index.shape[1]

    src = edge_index[0].astype(I32)
    rid = (edge_index[1] * 4 + edge_type).astype(I32)
    src_p = jnp.concatenate([src, jnp.zeros((E_PAD - e,), I32)])
    rid_p = jnp.concatenate([rid, jnp.full((E_PAD - e,), 4 * n, I32)])

    xf = x[:, 0].astype(F32)
    xp = jnp.pad(x.astype(F32), ((0, NPAD - n), (0, 0)))
    bf = batch.astype(F32)
    bcol = jnp.pad(bf, (0, NPAD - n), constant_values=PAD_G)[:, None]
    brow = bcol.reshape(1, NPAD)

    w1m = w1[:, 0, :].astype(F32)
    r1 = root1.astype(F32)
    b1r = b1.reshape(1, -1).astype(F32)
    w2s = w2.reshape(4 * 128, 256).astype(F32)
    b2r = b2.reshape(1, -1).astype(F32)
    expand = jnp.kron(jnp.eye(4, dtype=F32), jnp.ones((1, 128), F32))

    outv, outc = _edge_scalar_sc(xf, src_p, rid_p)
    svr = outv.reshape(NC, NPAD, 4)
    scr = outc.reshape(NC, NPAD, 4)

    z1, s1, s2, cg = _tc_layer1(xp, svr, scr, bcol, brow, w1m, r1, b1r)
    a1, b1n = _tc_norm_params(s1, s2, cg, g1w.reshape(1, -1),
                              g1b.reshape(1, -1), g1a.reshape(1, -1), 128)
    h1 = _tc_apply_relu(z1, bcol, a1, b1n)

    acc = _edge_rows_sc(h1, src_p, rid_p)
    accr = acc.reshape(RTOT_E // 4, 512)

    z2, s1b, s2b, cg2 = _tc_layer2(h1, accr, scr, bcol, brow, expand,
                                   root2.astype(F32), w2s, b2r)
    a2, b2n = _tc_norm_params(s1b, s2b, cg2, g2w.reshape(1, -1),
                              g2b.reshape(1, -1), g2a.reshape(1, -1), 256)

    return _tc_final(z2, bcol, brow, a2, b2n, cg2,
                     lin_w.astype(F32), lin_b.reshape(1, -1).astype(F32))


# speedup vs baseline: 6.2922x; 1.0711x over previous
import functools

import jax
import jax.numpy as jnp
from jax import lax
from jax.experimental import pallas as pl
from jax.experimental.pallas import tpu as pltpu
from jax.experimental.pallas import tpu_sc as plsc

F32 = jnp.float32
HI = jax.lax.Precision.HIGHEST
I32 = jnp.int32

NC = 2
NS = 16
LANES = 16
NW = NC * NS

G = 64
PAD_G = 64.0

NB = 1024
NPAD = 50176
NBLK = NPAD // NB

E_PAD = 819200
EPW = E_PAD // NW
CK = 1280
KG = 64
DKG = 2 * KG
CB = CK + DKG + LANES

RID_CAP = 4 * NPAD
NP = 9
SCROWS = 12160
ACC_ROWS = 12288
TROWS = ACC_ROWS // NS
ZROWS = 48
CROWS = SCROWS // NS
RTOT_E = NP * NC * SCROWS


def _edge_scalar_sc(xf, src_p, rid_p):
    mesh = plsc.VectorSubcoreMesh(
        core_axis_name="c", subcore_axis_name="s", num_cores=NC,
        num_subcores=NS)
    zslice = RID_CAP // NS

    @functools.partial(
        pl.kernel,
        out_type=(jax.ShapeDtypeStruct((NC, RID_CAP), F32),
                  jax.ShapeDtypeStruct((NC, RID_CAP), F32)),
        mesh=mesh,
        scratch_types=[
            pltpu.VMEM((CK,), I32),
            pltpu.VMEM((CK,), I32),
            pltpu.VMEM((CK,), F32),
            pltpu.VMEM((CK,), F32),
            pltpu.VMEM_SHARED((RID_CAP,), F32),
            pltpu.VMEM_SHARED((RID_CAP,), F32),
            pltpu.SemaphoreType.DMA,
        ],
    )
    def k(x_hbm, src_hbm, rid_hbm, z_hbm, outv_hbm, outc_hbm,
          src_v, rid_v, val_v, ones_v, accv, accc, sem):
        c = lax.axis_index("c")
        s = lax.axis_index("s")
        wid = s * NC + c

        pltpu.sync_copy(z_hbm.at[pl.ds(s * zslice, zslice)],
                        accv.at[pl.ds(s * zslice, zslice)])
        pltpu.sync_copy(z_hbm.at[pl.ds(s * zslice, zslice)],
                        accc.at[pl.ds(s * zslice, zslice)])

        def fill(i, _):
            ones_v[pl.ds(i * LANES, LANES)] = jnp.full((LANES,), 1.0, F32)
            return 0
        lax.fori_loop(0, CK // LANES, fill, 0)
        plsc.subcore_barrier()

        def step(i, _):
            base = wid * EPW + i * CK
            pltpu.sync_copy(src_hbm.at[pl.ds(base, CK)], src_v)
            pltpu.sync_copy(rid_hbm.at[pl.ds(base, CK)], rid_v)
            pltpu.async_copy(x_hbm.at[src_v], val_v, sem).wait()
            pltpu.sync_copy(val_v, accv.at[rid_v], add=True)
            pltpu.sync_copy(ones_v, accc.at[rid_v], add=True)
            return 0
        lax.fori_loop(0, EPW // CK, step, 0)
        plsc.subcore_barrier()

        pltpu.sync_copy(accv.at[pl.ds(s * zslice, zslice)],
                        outv_hbm.at[c, pl.ds(s * zslice, zslice)])
        pltpu.sync_copy(accc.at[pl.ds(s * zslice, zslice)],
                        outc_hbm.at[c, pl.ds(s * zslice, zslice)])

    zeros = jnp.zeros((RID_CAP,), F32)
    return k(xf, src_p, rid_p, zeros)


def _edge_rows_sc(h1, edges2):
    mesh = plsc.VectorSubcoreMesh(
        core_axis_name="c", subcore_axis_name="s", num_cores=NC,
        num_subcores=NS)

    @functools.partial(
        pl.kernel,
        out_type=jax.ShapeDtypeStruct((RTOT_E, 128), F32),
        mesh=mesh,
        scratch_types=[
            pltpu.VMEM((CB,), I32),
            pltpu.VMEM((CB,), I32),
            pltpu.VMEM((2, CK), I32),
            pltpu.VMEM((2, CK), I32),
            pltpu.VMEM((KG,), I32),
            pltpu.VMEM((KG,), I32),
            pltpu.VMEM((KG,), I32),
            pltpu.VMEM((KG,), I32),
            pltpu.VMEM((KG, 128), F32),
            pltpu.VMEM((KG, 128), F32),
            pltpu.VMEM((ZROWS, 128), F32),
            pltpu.VMEM_SHARED((ACC_ROWS, 128), F32),
            pltpu.SemaphoreType.DMA,
            pltpu.SemaphoreType.DMA,
            pltpu.SemaphoreType.DMA,
            pltpu.SemaphoreType.DMA,
        ],
        compiler_params=pltpu.CompilerParams(needs_layout_passes=False),
    )
    def k(h_hbm, e_hbm, out_hbm,
          src_c, lid_c, eb0, eb1, idxg0, idxs0, idxg1, idxs1,
          rows0, rows1, zrows, acc, sem0, sem1, semA, semB):
        c = lax.axis_index("c")
        s = lax.axis_index("s")
        ept = E_PAD // NS

        def zfill(i, _):
            for kk in range(128 // LANES):
                zrows[i, pl.ds(kk * LANES, LANES)] = (
                    jnp.zeros((LANES,), F32))
            return 0
        lax.fori_loop(0, ZROWS, zfill, 0)
        iot = lax.iota(I32, LANES)

        def drain(t, _):
            def cp(j, _):
                idxg0[pl.ds(j * LANES, LANES)] = (
                    src_c[pl.ds(t * DKG + j * LANES, LANES)])
                idxs0[pl.ds(j * LANES, LANES)] = (
                    lid_c[pl.ds(t * DKG + j * LANES, LANES)])
                idxg1[pl.ds(j * LANES, LANES)] = (
                    src_c[pl.ds(t * DKG + KG + j * LANES, LANES)])
                idxs1[pl.ds(j * LANES, LANES)] = (
                    lid_c[pl.ds(t * DKG + KG + j * LANES, LANES)])
                return 0
            lax.fori_loop(0, KG // LANES, cp, 0)
            d0 = pltpu.async_copy(h_hbm.at[idxg0], rows0, sem0)
            d1 = pltpu.async_copy(h_hbm.at[idxg1], rows1, sem1)
            d0.wait()
            pltpu.sync_copy(rows0, acc.at[idxs0], add=True)
            d1.wait()
            pltpu.sync_copy(rows1, acc.at[idxs1], add=True)
            return 0

        def pass_body(p, _):
            r0 = (p * NC + c) * SCROWS

            def zero_acc(kk, _):
                pltpu.sync_copy(
                    zrows, acc.at[pl.ds(s * TROWS + kk * ZROWS, ZROWS)])
                return 0
            lax.fori_loop(0, TROWS // ZROWS, zero_acc, 0)
            plsc.subcore_barrier()

            nchunks = (E_PAD // NS) // CK

            def stage_slice(i):
                ii = jnp.minimum(i, nchunks - 1)
                return e_hbm.at[:, pl.ds(s * ept + ii * CK, CK)]

            def compact(eb, off):
                def vec(j, off):
                    rv = eb[0, pl.ds(j * LANES, LANES)]
                    sv = eb[1, pl.ds(j * LANES, LANES)]
                    lid = rv - r0
                    m = (lid >= 0) & (lid < SCROWS)
                    cs = plsc.cumsum(jnp.where(m, 1.0, 0.0))
                    pos = off + cs.astype(I32) - 1
                    plsc.store_scatter(lid_c, [pos], lid, mask=m)
                    plsc.store_scatter(src_c, [pos], sv, mask=m)
                    return off + cs[LANES - 1].astype(I32)
                off = lax.fori_loop(0, CK // LANES, vec, off)

                nfull = off // DKG
                lax.fori_loop(0, nfull, drain, 0)
                rem = off - nfull * DKG

                def mv(kk, _):
                    lid_c[pl.ds(kk * LANES, LANES)] = (
                        lid_c[pl.ds(nfull * DKG + kk * LANES, LANES)])
                    src_c[pl.ds(kk * LANES, LANES)] = (
                        src_c[pl.ds(nfull * DKG + kk * LANES, LANES)])
                    return 0
                lax.fori_loop(0, (rem + LANES - 1) // LANES, mv, 0)
                return rem

            pltpu.async_copy(stage_slice(0), eb0, semA)

            def chunk2(i2, off):
                pltpu.make_async_copy(stage_slice(2 * i2), eb0, semA).wait()
                pltpu.async_copy(stage_slice(2 * i2 + 1), eb1, semB)
                off = compact(eb0, off)
                pltpu.make_async_copy(
                    stage_slice(2 * i2 + 1), eb1, semB).wait()
                pltpu.async_copy(stage_slice(2 * i2 + 2), eb0, semA)
                return compact(eb1, off)
            off = lax.fori_loop(0, nchunks // 2, chunk2, jnp.zeros((), I32))
            pltpu.make_async_copy(stage_slice(nchunks - 1), eb0, semA).wait()

            for kk in range(DKG // LANES):
                pos = off + kk * LANES + iot
                plsc.store_scatter(
                    lid_c, [pos], jnp.full((LANES,), SCROWS, I32))
                plsc.store_scatter(
                    src_c, [pos], jnp.zeros((LANES,), I32))
            lax.fori_loop(0, (off + DKG - 1) // DKG, drain, 0)
            plsc.subcore_barrier()

            pltpu.sync_copy(acc.at[pl.ds(s * CROWS, CROWS)],
                            out_hbm.at[pl.ds(r0 + s * CROWS, CROWS)])
            plsc.subcore_barrier()
            return 0
        lax.fori_loop(0, NP, pass_body, 0)

    return k(h1, edges2)


def _iota_row(width):
    return lax.broadcasted_iota(I32, (1, width), 1).astype(F32)


def _tc_layer1(xp, svr, scr, bcol, brow, w1m, r1, b1r):
    def body(x_ref, sv_ref, sc_ref, bc_ref, br_ref, w_ref, r_ref, b_ref,
             z_ref, s1_ref, s2_ref, cg_ref):
        i = pl.program_id(0)
        xb = x_ref[...]
        sv = sv_ref[0] + sv_ref[1]
        scnt = sc_ref[0] + sc_ref[1]
        mean = sv / jnp.maximum(scnt, 1.0)
        z = (xb * r_ref[...]
             + jnp.dot(mean, w_ref[...], preferred_element_type=F32, precision=HI)
             + b_ref[...])
        z_ref[...] = z
        oht = (br_ref[...] == lax.broadcasted_iota(I32, (G, NB), 0).astype(F32)
               ).astype(F32)
        s1c = jnp.dot(oht, z, preferred_element_type=F32, precision=HI)
        s2c = jnp.dot(oht, z * z, preferred_element_type=F32, precision=HI)
        cgc = jnp.dot(oht, jnp.ones_like(z), preferred_element_type=F32, precision=HI)

        @pl.when(i == 0)
        def _():
            s1_ref[...] = jnp.zeros_like(s1_ref)
            s2_ref[...] = jnp.zeros_like(s2_ref)
            cg_ref[...] = jnp.zeros_like(cg_ref)
        s1_ref[...] += s1c
        s2_ref[...] += s2c
        cg_ref[...] += cgc

    full = lambda shape: pl.BlockSpec(shape, lambda i: tuple(0 for _ in shape))
    return pl.pallas_call(
        body,
        grid=(NBLK,),
        in_specs=[
            pl.BlockSpec((NB, 1), lambda i: (i, 0)),
            pl.BlockSpec((2, NB, 4), lambda i: (0, i, 0)),
            pl.BlockSpec((2, NB, 4), lambda i: (0, i, 0)),
            pl.BlockSpec((NB, 1), lambda i: (i, 0)),
            pl.BlockSpec((1, NB), lambda i: (0, i)),
            full((4, 128)),
            full((1, 128)),
            full((1, 128)),
        ],
        out_specs=[
            pl.BlockSpec((NB, 128), lambda i: (i, 0)),
            full((G, 128)),
            full((G, 128)),
            full((G, 128)),
        ],
        out_shape=[
            jax.ShapeDtypeStruct((NPAD, 128), F32),
            jax.ShapeDtypeStruct((G, 128), F32),
            jax.ShapeDtypeStruct((G, 128), F32),
            jax.ShapeDtypeStruct((G, 128), F32),
        ],
    )(xp, svr, scr, bcol, brow, w1m, r1, b1r)


def _tc_norm_params(s1, s2, cg, gw, gb, ga, width):
    def body(s1_ref, s2_ref, cg_ref, gw_ref, gb_ref, ga_ref, a_ref, b_ref):
        c = jnp.maximum(cg_ref[...], 1.0)
        m = s1_ref[...] / c
        a = ga_ref[...]
        var = s2_ref[...] / c - (2.0 * a - a * a) * m * m
        aa = gw_ref[...] * lax.rsqrt(var + 1e-5)
        a_ref[...] = aa
        b_ref[...] = gb_ref[...] - aa * a * m

    full = lambda shape: pl.BlockSpec(shape, lambda: tuple(0 for _ in shape))
    return pl.pallas_call(
        body,
        in_specs=[full((G, width))] * 3 + [full((1, width))] * 3,
        out_specs=[full((G, width))] * 2,
        out_shape=[jax.ShapeDtypeStruct((G, width), F32)] * 2,
    )(s1, s2, cg, gw, gb, ga)


def _tc_apply_relu(z1, bcol, a1, b1n):
    def body(z_ref, bc_ref, a_ref, b_ref, h_ref):
        oh = (bc_ref[...] == _iota_row(G)).astype(F32)
        an = jnp.dot(oh, a_ref[...], preferred_element_type=F32, precision=HI)
        bn = jnp.dot(oh, b_ref[...], preferred_element_type=F32, precision=HI)
        h_ref[...] = jnp.maximum(an * z_ref[...] + bn, 0.0)

    full = lambda shape: pl.BlockSpec(shape, lambda i: tuple(0 for _ in shape))
    return pl.pallas_call(
        body,
        grid=(NBLK,),
        in_specs=[
            pl.BlockSpec((NB, 128), lambda i: (i, 0)),
            pl.BlockSpec((NB, 1), lambda i: (i, 0)),
            full((G, 128)),
            full((G, 128)),
        ],
        out_specs=pl.BlockSpec((NB, 128), lambda i: (i, 0)),
        out_shape=jax.ShapeDtypeStruct((NPAD, 128), F32),
    )(z1, bcol, a1, b1n)


def _tc_layer2(h1, accr, scr, bcol, brow, expand, r2, w2s, b2r):
    def body(h_ref, acc_ref, sc_ref, bc_ref, br_ref, e_ref, r_ref, w_ref,
             b_ref, z_ref, s1_ref, s2_ref, cg_ref):
        i = pl.program_id(0)
        cnt = sc_ref[0] + sc_ref[1]
        recip = 1.0 / jnp.maximum(cnt, 1.0)
        rb = jnp.dot(recip, e_ref[...], preferred_element_type=F32, precision=HI)
        mean = acc_ref[...] * rb
        z = (jnp.dot(h_ref[...], r_ref[...], preferred_element_type=F32)
             + jnp.dot(mean, w_ref[...], preferred_element_type=F32)
             + b_ref[...])
        z_ref[...] = z
        oht = (br_ref[...] == lax.broadcasted_iota(I32, (G, NB), 0).astype(F32)
               ).astype(F32)
        s1c = jnp.dot(oht, z, preferred_element_type=F32, precision=HI)
        s2c = jnp.dot(oht, z * z, preferred_element_type=F32, precision=HI)
        cgc = jnp.dot(oht, jnp.ones_like(z), preferred_element_type=F32, precision=HI)

        @pl.when(i == 0)
        def _():
            s1_ref[...] = jnp.zeros_like(s1_ref)
            s2_ref[...] = jnp.zeros_like(s2_ref)
            cg_ref[...] = jnp.zeros_like(cg_ref)
        s1_ref[...] += s1c
        s2_ref[...] += s2c
        cg_ref[...] += cgc

    full = lambda shape: pl.BlockSpec(shape, lambda i: tuple(0 for _ in shape))
    return pl.pallas_call(
        body,
        grid=(NBLK,),
        in_specs=[
            pl.BlockSpec((NB, 128), lambda i: (i, 0)),
            pl.BlockSpec((NB, 512), lambda i: (i, 0)),
            pl.BlockSpec((2, NB, 4), lambda i: (0, i, 0)),
            pl.BlockSpec((NB, 1), lambda i: (i, 0)),
            pl.BlockSpec((1, NB), lambda i: (0, i)),
            full((4, 512)),
            full((128, 256)),
            full((512, 256)),
            full((1, 256)),
        ],
        out_specs=[
            pl.BlockSpec((NB, 256), lambda i: (i, 0)),
            full((G, 256)),
            full((G, 256)),
            full((G, 256)),
        ],
        out_shape=[
            jax.ShapeDtypeStruct((NPAD, 256), F32),
            jax.ShapeDtypeStruct((G, 256), F32),
            jax.ShapeDtypeStruct((G, 256), F32),
            jax.ShapeDtypeStruct((G, 256), F32),
        ],
    )(h1, accr, scr, bcol, brow, expand, r2, w2s, b2r)


def _tc_final(z2, bcol, brow, a2, b2n, cg2, lw, lb):
    def body(z_ref, bc_ref, br_ref, a_ref, b_ref, cg_ref, lw_ref, lb_ref,
             out_ref, p_acc):
        i = pl.program_id(0)

        @pl.when(i == 0)
        def _():
            p_acc[...] = jnp.zeros_like(p_acc)
        oh = (bc_ref[...] == _iota_row(G)).astype(F32)
        an = jnp.dot(oh, a_ref[...], preferred_element_type=F32, precision=HI)
        bn = jnp.dot(oh, b_ref[...], preferred_element_type=F32, precision=HI)
        h = jnp.maximum(an * z_ref[...] + bn, 0.0)
        oht = (br_ref[...] == lax.broadcasted_iota(I32, (G, NB), 0).astype(F32)
               ).astype(F32)
        p_acc[...] += jnp.dot(oht, h, preferred_element_type=F32, precision=HI)

        @pl.when(i == NBLK - 1)
        def _():
            pooled = p_acc[...] / jnp.maximum(cg_ref[...], 1.0)
            out_ref[...] = (jnp.dot(pooled, lw_ref[...],
                                    preferred_element_type=F32, precision=HI)
                            + lb_ref[...])

    full = lambda shape: pl.BlockSpec(shape, lambda i: tuple(0 for _ in shape))
    return pl.pallas_call(
        body,
        grid=(NBLK,),
        in_specs=[
            pl.BlockSpec((NB, 256), lambda i: (i, 0)),
            pl.BlockSpec((NB, 1), lambda i: (i, 0)),
            pl.BlockSpec((1, NB), lambda i: (0, i)),
            full((G, 256)),
            full((G, 256)),
            full((G, 256)),
            full((256, 10)),
            full((1, 10)),
        ],
        out_specs=full((G, 10)),
        out_shape=jax.ShapeDtypeStruct((G, 10), F32),
        scratch_shapes=[pltpu.VMEM((G, 256), F32)],
    )(z2, bcol, brow, a2, b2n, cg2, lw, lb)


def kernel(x, edge_index, edge_type, batch, w1, root1, b1, w2, root2, b2,
           g1w, g1b, g1a, g2w, g2b, g2a, lin_w, lin_b):
    n = x.shape[0]
    e = edge_index.shape[1]

    src = edge_index[0].astype(I32)
    rid = (edge_index[1] * 4 + edge_type).astype(I32)
    src_p = jnp.concatenate([src, jnp.zeros((E_PAD - e,), I32)])
    rid_p = jnp.concatenate([rid, jnp.full((E_PAD - e,), 4 * n, I32)])

    xf = x[:, 0].astype(F32)
    xp = jnp.pad(x.astype(F32), ((0, NPAD - n), (0, 0)))
    bf = batch.astype(F32)
    bcol = jnp.pad(bf, (0, NPAD - n), constant_values=PAD_G)[:, None]
    brow = bcol.reshape(1, NPAD)

    w1m = w1[:, 0, :].astype(F32)
    r1 = root1.astype(F32)
    b1r = b1.reshape(1, -1).astype(F32)
    w2s = w2.reshape(4 * 128, 256).astype(F32)
    b2r = b2.reshape(1, -1).astype(F32)
    expand = jnp.kron(jnp.eye(4, dtype=F32), jnp.ones((1, 128), F32))

    outv, outc = _edge_scalar_sc(xf, src_p, rid_p)
    svr = outv.reshape(NC, NPAD, 4)
    scr = outc.reshape(NC, NPAD, 4)

    z1, s1, s2, cg = _tc_layer1(xp, svr, scr, bcol, brow, w1m, r1, b1r)
    a1, b1n = _tc_norm_params(s1, s2, cg, g1w.reshape(1, -1),
                              g1b.reshape(1, -1), g1a.reshape(1, -1), 128)
    h1 = _tc_apply_relu(z1, bcol, a1, b1n)

    acc = _edge_rows_sc(h1, jnp.stack([rid_p, src_p]))
    accr = acc.reshape(RTOT_E // 4, 512)

    z2, s1b, s2b, cg2 = _tc_layer2(h1, accr, scr, bcol, brow, expand,
                                   root2.astype(F32), w2s, b2r)
    a2, b2n = _tc_norm_params(s1b, s2b, cg2, g2w.reshape(1, -1),
                              g2b.reshape(1, -1), g2a.reshape(1, -1), 256)

    return _tc_final(z2, bcol, brow, a2, b2n, cg2,
                     lin_w.astype(F32), lin_b.reshape(1, -1).astype(F32))
